# baseline (speedup 1.0000x reference)
import functools

import jax
import jax.numpy as jnp
from jax import lax
from jax.experimental import pallas as pl
from jax.experimental.pallas import tpu as pltpu
from jax.experimental.pallas import tpu_sc as plsc

B, C, T = 4, 256, 2048
TT = 128
K = 6
F32 = jnp.float32
I32 = jnp.int32

_NC = 2
_NS = 16
_CHUNK = (B * T) // (_NC * _NS)
_CPB = T // _CHUNK


def _prep_body(x_ref, D_ref, te_ref, Wtd_ref, P_ref,
               stats_ref, tf_ref, t2_ref):
    b = pl.program_id(0)
    xb = x_ref[0]
    zcol = jnp.zeros((C, 1), F32)
    xpad = jnp.concatenate([zcol, xb, zcol], axis=1)
    acc = None
    for s in range(3):
        d = lax.dot_general(D_ref[s], xpad[:, s:s + T],
                            (((1,), (0,)), ((), ())), preferred_element_type=F32)
        acc = d if acc is None else acc + d
    y = acc + P_ref[2][:, None]

    @pl.when(b == 0)
    def _():
        stats_ref[...] = jnp.zeros_like(stats_ref)

    stats_ref[0, :] += jnp.sum(y, axis=1)
    stats_ref[1, :] += jnp.sum(y * y, axis=1)

    te = te_ref[...]
    yts = []
    ts1 = jnp.zeros((C,), F32)
    ts2 = jnp.zeros((C,), F32)
    for bb in range(B):
        yt = lax.dot_general(Wtd_ref[...], te[bb],
                             (((1,), (1,)), ((), ())), preferred_element_type=F32)
        yt = yt + P_ref[5][:, None]
        yts.append(yt)
        ts1 = ts1 + jnp.sum(yt, axis=1)
        ts2 = ts2 + jnp.sum(yt * yt, axis=1)
    nmt = float(B * TT)
    meant = ts1 / nmt
    vart = ts2 / nmt - meant * meant
    scalet = P_ref[6] * lax.rsqrt(vart + 1e-5)
    shiftt = P_ref[7] - meant * scalet
    for bb in range(B):
        tf = jnp.maximum(yts[bb] * scalet[:, None] + shiftt[:, None], 0.0)
        tf_ref[bb] = tf
        t2_ref[bb] = jnp.broadcast_to(jnp.sum(tf * tf, axis=0)[None, :], (8, TT))


def _scores_t(xc, tf, t2):
    crossT = lax.dot_general(tf, xc, (((0,), (0,)), ((), ())),
                             preferred_element_type=F32)
    x2 = jnp.sum(xc * xc, axis=0)
    return 2.0 * crossT - x2[None, :] - t2[:, None]


def _stage_a_body(x_ref, D_ref, stats_ref, P_ref, tf_ref, t2_ref,
                  xn_ref, sc_ref):
    xb = x_ref[0]
    zcol = jnp.zeros((C, 1), F32)
    xpad = jnp.concatenate([zcol, xb, zcol], axis=1)
    acc = None
    for s in range(3):
        d = lax.dot_general(D_ref[s], xpad[:, s:s + T],
                            (((1,), (0,)), ((), ())), preferred_element_type=F32)
        acc = d if acc is None else acc + d
    y = acc + P_ref[2][:, None]
    n = float(B * T)
    mean = stats_ref[0] / n
    var = stats_ref[1] / n - mean * mean
    scale = P_ref[0] * lax.rsqrt(var + 1e-5)
    shift = P_ref[1] - mean * scale
    xn = jnp.maximum(y * scale[:, None] + shift[:, None], 0.0)
    xn_ref[0] = xn
    sc_ref[0] = _scores_t(xn, tf_ref[0], t2_ref[0, 0, :])


def _gcn_combine(xc, idx, tf, A, Wd, bg):
    iota = lax.broadcasted_iota(I32, (T, TT), 1)
    agg = jnp.full((C // 2, T), -jnp.inf, F32)
    for k in range(K):
        oh = (iota == idx[k][:, None]).astype(F32)
        nbr = lax.dot_general(tf, oh, (((1,), (1,)), ((), ())),
                              preferred_element_type=F32,
                              precision=lax.Precision.HIGHEST)
        v = lax.dot_general(Wd, nbr - xc, (((1,), (0,)), ((), ())),
                            preferred_element_type=F32)
        agg = jnp.maximum(agg, v)
    lo = lax.dot_general(A, xc, (((1,), (0,)), ((), ())),
                         preferred_element_type=F32) + bg[:128][:, None]
    hi = agg + bg[128:][:, None]
    return jnp.maximum(xc + jnp.concatenate([lo, hi], axis=0), 0.0)


def _stage_b_body(xn_ref, idx_ref, tf_ref, t2_ref, A1_ref, Wd1_ref, P_ref,
                  x1_ref, sc_ref):
    x1 = _gcn_combine(xn_ref[0], idx_ref[0], tf_ref[0],
                      A1_ref[...], Wd1_ref[...], P_ref[3])
    x1_ref[0] = x1
    sc_ref[0] = _scores_t(x1, tf_ref[0], t2_ref[0, 0, :])


def _stage_c_body(x_ref, x1_ref, idx_ref, tf_ref, A2_ref, Wd2_ref, P_ref,
                  out_ref):
    xg = _gcn_combine(x1_ref[0], idx_ref[0], tf_ref[0],
                      A2_ref[...], Wd2_ref[...], P_ref[4])
    out_ref[0] = xg + x_ref[0]


def _sc_topk_body(sc_hbm, out_hbm, sv, ov):
    wid = lax.axis_index("s") * _NC + lax.axis_index("c")
    b = wid // _CPB
    base = (wid % _CPB) * _CHUNK
    pltpu.sync_copy(sc_hbm.at[b, :, pl.ds(base, _CHUNK)], sv)

    neg_inf = jnp.full((16,), -jnp.inf, F32)
    zero_i = jnp.zeros((16,), I32)

    def group_body(g, carry_g):
        def topic_body(s, carry):
            ms = list(carry[:K])
            ids = list(carry[K:])
            tv = sv[s, pl.ds(g * 16, 16)]
            ti = jnp.full((16,), s, I32)
            for j in range(K):
                gt = tv > ms[j]
                nm = jnp.where(gt, tv, ms[j])
                ni = jnp.where(gt, ti, ids[j])
                tv = jnp.where(gt, ms[j], tv)
                ti = jnp.where(gt, ids[j], ti)
                ms[j] = nm
                ids[j] = ni
            return tuple(ms) + tuple(ids)

        init = tuple([neg_inf] * K) + tuple([zero_i] * K)
        fin = lax.fori_loop(0, TT, topic_body, init)
        for k in range(K):
            ov[k, pl.ds(g * 16, 16)] = fin[K + k]
        ov[K, pl.ds(g * 16, 16)] = zero_i
        ov[K + 1, pl.ds(g * 16, 16)] = zero_i
        return carry_g

    lax.fori_loop(0, _CHUNK // 16, group_body, 0)
    pltpu.sync_copy(ov, out_hbm.at[b, :, pl.ds(base, _CHUNK)])


_sc_topk = functools.partial(
    pl.kernel,
    mesh=plsc.VectorSubcoreMesh(core_axis_name="c", subcore_axis_name="s"),
    out_type=jax.ShapeDtypeStruct((B, 8, T), I32),
    scratch_types=[
        pltpu.VMEM((TT, _CHUNK), F32),
        pltpu.VMEM((8, _CHUNK), I32),
    ],
)(_sc_topk_body)


def _const(shape):
    nd = len(shape)
    return pl.BlockSpec(shape, lambda b: (0,) * nd)


def _perb(shape):
    nd = len(shape)
    return pl.BlockSpec((1,) + shape[1:], lambda b: (b,) + (0,) * (nd - 1))


def kernel(snip_features, topic_embedding, W1, b1, gamma1, beta1,
           Wt, bt, gammat, betat, Wg1, bg1, Wg2, bg2):
    eye4 = jnp.eye(4, dtype=F32)
    eye16 = jnp.eye(16, dtype=F32)
    W1r = W1.reshape(4, 64, 64, 3)
    D = jnp.einsum('gois,gh->sgohi', W1r, eye4).reshape(3, C, C)
    Wtd = jnp.einsum('goi,gh->gohi', Wt[:, :, 0].reshape(4, 64, 4), eye4).reshape(C, 16)
    A1 = jnp.einsum('gdc,gh->gdhc', Wg1[:16], eye16).reshape(C // 2, C)
    Wd1 = jnp.einsum('gdc,gh->gdhc', Wg1[16:], eye16).reshape(C // 2, C)
    A2 = jnp.einsum('gdc,gh->gdhc', Wg2[:16], eye16).reshape(C // 2, C)
    Wd2 = jnp.einsum('gdc,gh->gdhc', Wg2[16:], eye16).reshape(C // 2, C)
    P = jnp.stack([gamma1, beta1, b1, bg1, bg2, bt, gammat, betat])

    stats, tfa, t2 = pl.pallas_call(
        _prep_body,
        grid=(B,),
        in_specs=[
            _perb((B, C, T)),
            _const((3, C, C)),
            _const((B, TT, 16)),
            _const((C, 16)),
            _const((8, C)),
        ],
        out_specs=[
            _const((8, C)),
            _const((B, C, TT)),
            _const((B, 8, TT)),
        ],
        out_shape=[
            jax.ShapeDtypeStruct((8, C), F32),
            jax.ShapeDtypeStruct((B, C, TT), F32),
            jax.ShapeDtypeStruct((B, 8, TT), F32),
        ],
    )(snip_features, D, topic_embedding, Wtd, P)

    xn, sc1 = pl.pallas_call(
        _stage_a_body,
        grid=(B,),
        in_specs=[
            _perb((B, C, T)),
            _const((3, C, C)),
            _const((8, C)),
            _const((8, C)),
            _perb((B, C, TT)),
            _perb((B, 8, TT)),
        ],
        out_specs=[_perb((B, C, T)), _perb((B, TT, T))],
        out_shape=[
            jax.ShapeDtypeStruct((B, C, T), F32),
            jax.ShapeDtypeStruct((B, TT, T), F32),
        ],
    )(snip_features, D, stats, P, tfa, t2)

    idx1 = _sc_topk(sc1)

    x1, sc2 = pl.pallas_call(
        _stage_b_body,
        grid=(B,),
        in_specs=[
            _perb((B, C, T)),
            _perb((B, 8, T)),
            _perb((B, C, TT)),
            _perb((B, 8, TT)),
            _const((C // 2, C)),
            _const((C // 2, C)),
            _const((8, C)),
        ],
        out_specs=[_perb((B, C, T)), _perb((B, TT, T))],
        out_shape=[
            jax.ShapeDtypeStruct((B, C, T), F32),
            jax.ShapeDtypeStruct((B, TT, T), F32),
        ],
    )(xn, idx1, tfa, t2, A1, Wd1, P)

    idx2 = _sc_topk(sc2)

    out = pl.pallas_call(
        _stage_c_body,
        grid=(B,),
        in_specs=[
            _perb((B, C, T)),
            _perb((B, C, T)),
            _perb((B, 8, T)),
            _perb((B, C, TT)),
            _const((C // 2, C)),
            _const((C // 2, C)),
            _const((8, C)),
        ],
        out_specs=_perb((B, C, T)),
        out_shape=jax.ShapeDtypeStruct((B, C, T), F32),
    )(snip_features, x1, idx2, tfa, A2, Wd2, P)
    return out
